# Initial kernel scaffold; baseline (speedup 1.0000x reference)
import jax
import jax.numpy as jnp
from jax.experimental import pallas as pl


def kernel(x, edge_index, pre_W1, pre_b1, pre_W2, pre_b2, comb_W, comb_b, jk_W, jk_b, neigh_alphas, aggr_alphas, norm_alphas, comb_alphas, jk_alphas):
    raise NotImplementedError("write your pallas kernel here")


# speedup vs baseline: 29.9075x; 29.9075x over previous
import functools

import jax
import jax.numpy as jnp
from jax import lax
from jax.experimental import pallas as pl
from jax.experimental.pallas import tpu as pltpu
from jax.experimental.pallas import tpu_sc as plsc

N = 10000
E = 160000
F = 128
HID = 256
C = 40
DP = 48
NLAYERS = 2
TEMP = 0.5

NC, NS = 2, 16
NW = NC * NS
EC = 128
EPAD = 163840
TPT = EPAD // NW // EC
NPAD = 10112
RPT = NPAD // NS
BR = 1000
SEG = 1600


def _sc_mesh():
    return plsc.VectorSubcoreMesh(core_axis_name="c", subcore_axis_name="s",
                                  num_cores=NC, num_subcores=NS)


def _sc_prop_sum(h_pad, src2d, dst2d, zrows):
    @functools.partial(
        pl.kernel,
        out_type=jax.ShapeDtypeStruct((NC, NPAD, DP), jnp.float32),
        mesh=_sc_mesh(),
        scratch_types=[
            pltpu.VMEM((TPT, EC), jnp.int32),
            pltpu.VMEM((TPT, EC), jnp.int32),
            pltpu.VMEM((EC, DP), jnp.float32),
            pltpu.VMEM_SHARED((NPAD, DP), jnp.float32),
            pltpu.SemaphoreType.DMA,
        ],
        compiler_params=pltpu.CompilerParams(use_tc_tiling_on_sc=False),
    )
    def kfn(h_hbm, s_hbm, d_hbm, z_hbm, out_hbm, sidx, didx, rows, acc, sem):
        c = lax.axis_index("c")
        s = lax.axis_index("s")
        pltpu.sync_copy(z_hbm, acc.at[pl.ds(s * RPT, RPT)])
        tb = (c * NS + s) * TPT
        pltpu.sync_copy(s_hbm.at[pl.ds(tb, TPT)], sidx)
        pltpu.sync_copy(d_hbm.at[pl.ds(tb, TPT)], didx)
        plsc.subcore_barrier()

        def body(j, carry):
            pltpu.async_copy(h_hbm.at[sidx.at[j]], rows, sem).wait()
            pltpu.sync_copy(rows, acc.at[didx.at[j]], add=True)
            return carry

        lax.fori_loop(0, TPT, body, 0)
        plsc.subcore_barrier()
        pltpu.sync_copy(acc.at[pl.ds(s * RPT, RPT)],
                        out_hbm.at[c, pl.ds(s * RPT, RPT)])

    return kfn(h_pad, src2d, dst2d, zrows)


def _sc_degree(dst2d, orows, z16):
    @functools.partial(
        pl.kernel,
        out_type=jax.ShapeDtypeStruct((NC, NPAD, 16), jnp.float32),
        mesh=_sc_mesh(),
        scratch_types=[
            pltpu.VMEM((TPT, EC), jnp.int32),
            pltpu.VMEM((EC, 16), jnp.float32),
            pltpu.VMEM_SHARED((NPAD, 16), jnp.float32),
        ],
        compiler_params=pltpu.CompilerParams(use_tc_tiling_on_sc=False),
    )
    def kfn(d_hbm, o_hbm, z_hbm, out_hbm, didx, ones, acc):
        c = lax.axis_index("c")
        s = lax.axis_index("s")
        pltpu.sync_copy(z_hbm, acc.at[pl.ds(s * RPT, RPT)])
        pltpu.sync_copy(o_hbm, ones)
        tb = (c * NS + s) * TPT
        pltpu.sync_copy(d_hbm.at[pl.ds(tb, TPT)], didx)
        plsc.subcore_barrier()

        def body(j, carry):
            pltpu.sync_copy(ones, acc.at[didx.at[j]], add=True)
            return carry

        lax.fori_loop(0, TPT, body, 0)
        plsc.subcore_barrier()
        pltpu.sync_copy(acc.at[pl.ds(s * RPT, RPT)],
                        out_hbm.at[c, pl.ds(s * RPT, RPT)])

    return kfn(dst2d, orows, z16)


def _tc_premlp(x, W1, b1, W2, b2):
    def body(x_ref, w1_ref, b1_ref, w2_ref, b2_ref, o_ref):
        a = jnp.maximum(
            jnp.dot(x_ref[...], w1_ref[...],
                    preferred_element_type=jnp.float32) + b1_ref[...], 0.0)
        o_ref[...] = jnp.dot(a, w2_ref[...],
                             preferred_element_type=jnp.float32) + b2_ref[...]

    return pl.pallas_call(
        body,
        grid=(N // BR,),
        in_specs=[
            pl.BlockSpec((BR, F), lambda i: (i, 0)),
            pl.BlockSpec((F, HID), lambda i: (0, 0)),
            pl.BlockSpec((1, HID), lambda i: (0, 0)),
            pl.BlockSpec((HID, C), lambda i: (0, 0)),
            pl.BlockSpec((1, C), lambda i: (0, 0)),
        ],
        out_specs=pl.BlockSpec((BR, C), lambda i: (i, 0)),
        out_shape=jax.ShapeDtypeStruct((N, C), jnp.float32),
    )(x, W1, b1.reshape(1, HID), W2, b2.reshape(1, C))


def _tc_prep(degp, h, wv):
    def body(wv_ref, dp_ref, h_ref, sc_ref, hs_ref):
        deg = dp_ref[0, :, 0:1] + dp_ref[1, :, 0:1]
        pos = deg > 0.0
        dmax = jnp.maximum(deg, 1e-12)
        dis = jnp.where(pos, lax.rsqrt(dmax), 0.0)
        dinv = jnp.where(pos, 1.0 / dmax, 0.0)
        inv1 = 1.0 / jnp.maximum(deg, 1.0)
        one = jnp.ones_like(deg)
        cols = []
        for l in range(NLAYERS):
            sym = wv_ref[2 * l] > 0.5
            mean = wv_ref[2 * l + 1] > 0.5
            pre = jnp.where(sym, dis, one)
            post = jnp.where(sym, dis, dinv) * jnp.where(mean, inv1, one)
            cols += [pre, post, post * pre]
        z = jnp.zeros_like(deg)
        sc_ref[...] = jnp.concatenate(cols + [z, z], axis=1)
        hs_ref[...] = jnp.concatenate(
            [h_ref[...] * cols[0], jnp.zeros((BR, DP - C), jnp.float32)],
            axis=1)

    return pl.pallas_call(
        body,
        grid=(N // BR,),
        in_specs=[
            pl.BlockSpec(memory_space=pltpu.SMEM),
            pl.BlockSpec((NC, BR, 16), lambda i: (0, i, 0)),
            pl.BlockSpec((BR, C), lambda i: (i, 0)),
        ],
        out_specs=[
            pl.BlockSpec((BR, 8), lambda i: (i, 0)),
            pl.BlockSpec((BR, DP), lambda i: (i, 0)),
        ],
        out_shape=[
            jax.ShapeDtypeStruct((N, 8), jnp.float32),
            jax.ShapeDtypeStruct((N, DP), jnp.float32),
        ],
    )(wv, degp, h)


def _tc_segmax(hs, src2, dst2):
    def body(src_ref, dst_ref, hs_ref, o_ref):
        @pl.when(pl.program_id(0) == 0)
        def _():
            o_ref[...] = jnp.full((N, DP), -jnp.inf, jnp.float32)

        def step(e, carry):
            sv = src_ref[0, 0, e]
            dv = dst_ref[0, 0, e]
            row = hs_ref[pl.ds(sv, 1), :]
            o_ref[pl.ds(dv, 1), :] = jnp.maximum(o_ref[pl.ds(dv, 1), :], row)
            return carry

        lax.fori_loop(0, SEG, step, 0)

    return pl.pallas_call(
        body,
        grid=(E // SEG,),
        in_specs=[
            pl.BlockSpec((1, 1, SEG), lambda i: (i, 0, 0),
                         memory_space=pltpu.SMEM),
            pl.BlockSpec((1, 1, SEG), lambda i: (i, 0, 0),
                         memory_space=pltpu.SMEM),
            pl.BlockSpec((N, DP), lambda i: (0, 0)),
        ],
        out_specs=pl.BlockSpec((N, DP), lambda i: (0, 0)),
        out_shape=jax.ShapeDtypeStruct((N, DP), jnp.float32),
    )(src2, dst2, hs)


def _tc_mid(p_in, scales, l, is_max):
    mid_col = 3 * l + 2

    def body(p_ref, sc_ref, o_ref):
        if is_max:
            p48 = p_ref[...]
            p48 = jnp.where(jnp.isfinite(p48), p48, 0.0)
        else:
            p48 = p_ref[0] + p_ref[1]
        o_ref[...] = p48 * sc_ref[:, mid_col:mid_col + 1]

    p_spec = (pl.BlockSpec((BR, DP), lambda i: (i, 0)) if is_max
              else pl.BlockSpec((NC, BR, DP), lambda i: (0, i, 0)))
    return pl.pallas_call(
        body,
        grid=(N // BR,),
        in_specs=[p_spec, pl.BlockSpec((BR, 8), lambda i: (i, 0))],
        out_specs=pl.BlockSpec((BR, DP), lambda i: (i, 0)),
        out_shape=jax.ShapeDtypeStruct((N, DP), jnp.float32),
    )(p_in, scales)


def _tc_combine(p_in, xprev, scales, Wl, bl, wv, l, is_max, emit_scaled):
    post_col = 3 * l + 1

    def body(wv_ref, p_ref, xp_ref, sc_ref, w_ref, b_ref, *outs):
        if is_max:
            p48 = p_ref[...]
            p48 = jnp.where(jnp.isfinite(p48), p48, 0.0)
        else:
            p48 = p_ref[0] + p_ref[1]
        p = p48[:, :C] * sc_ref[:, post_col:post_col + 1]
        m = jnp.maximum(wv_ref[0] * p, 0.0)
        xp = xp_ref[...]
        cadd = m + xp
        ccat = (jnp.dot(m, w_ref[0:C, :], preferred_element_type=jnp.float32)
                + jnp.dot(xp, w_ref[C:, :], preferred_element_type=jnp.float32)
                + b_ref[...])
        xn = wv_ref[1] * cadd + wv_ref[2] * ccat
        outs[0][...] = xn
        if emit_scaled:
            outs[1][...] = jnp.concatenate(
                [xn * sc_ref[:, 3:4], jnp.zeros((BR, DP - C), jnp.float32)],
                axis=1)

    p_spec = (pl.BlockSpec((BR, DP), lambda i: (i, 0)) if is_max
              else pl.BlockSpec((NC, BR, DP), lambda i: (0, i, 0)))
    out_specs = [pl.BlockSpec((BR, C), lambda i: (i, 0))]
    out_shape = [jax.ShapeDtypeStruct((N, C), jnp.float32)]
    if emit_scaled:
        out_specs.append(pl.BlockSpec((BR, DP), lambda i: (i, 0)))
        out_shape.append(jax.ShapeDtypeStruct((N, DP), jnp.float32))
    res = pl.pallas_call(
        body,
        grid=(N // BR,),
        in_specs=[
            pl.BlockSpec(memory_space=pltpu.SMEM),
            p_spec,
            pl.BlockSpec((BR, C), lambda i: (i, 0)),
            pl.BlockSpec((BR, 8), lambda i: (i, 0)),
            pl.BlockSpec((2 * C, C), lambda i: (0, 0)),
            pl.BlockSpec((1, C), lambda i: (0, 0)),
        ],
        out_specs=out_specs,
        out_shape=out_shape,
    )(wv, p_in, xprev, scales, Wl, bl.reshape(1, C))
    return res if emit_scaled else (res[0], None)


def _tc_jk(h0, x1, x2, jk_W, jk_b, jw):
    def body(jw_ref, h_ref, x1_ref, x2_ref, w_ref, b_ref, o_ref):
        h = h_ref[...]
        x1 = x1_ref[...]
        x2 = x2_ref[...]
        cat = (jnp.dot(h, w_ref[0:C, :], preferred_element_type=jnp.float32)
               + jnp.dot(x1, w_ref[C:2 * C, :],
                         preferred_element_type=jnp.float32)
               + jnp.dot(x2, w_ref[2 * C:, :],
                         preferred_element_type=jnp.float32)
               + b_ref[...])
        mx = jnp.maximum(jnp.maximum(h, x1), x2)
        mn = (h + x1 + x2) / 3.0
        lin = (jw_ref[0] * x2 + jw_ref[1] * mx + jw_ref[2] * mn
               + jw_ref[3] * cat)
        rmax = jnp.max(lin, axis=1, keepdims=True)
        sh = lin - rmax
        o_ref[...] = sh - jnp.log(jnp.sum(jnp.exp(sh), axis=1, keepdims=True))

    return pl.pallas_call(
        body,
        grid=(N // BR,),
        in_specs=[
            pl.BlockSpec(memory_space=pltpu.SMEM),
            pl.BlockSpec((BR, C), lambda i: (i, 0)),
            pl.BlockSpec((BR, C), lambda i: (i, 0)),
            pl.BlockSpec((BR, C), lambda i: (i, 0)),
            pl.BlockSpec((3 * C, C), lambda i: (0, 0)),
            pl.BlockSpec((1, C), lambda i: (0, 0)),
        ],
        out_specs=pl.BlockSpec((BR, C), lambda i: (i, 0)),
        out_shape=jax.ShapeDtypeStruct((N, C), jnp.float32),
    )(jw, h0, x1, x2, jk_W, jk_b.reshape(1, C))


def _categ(alphas, u):
    ws = jax.nn.softmax((alphas - jnp.log(-jnp.log(u))) / TEMP, axis=-1)
    oh = jax.nn.one_hot(jnp.argmax(ws, axis=-1), ws.shape[-1], dtype=ws.dtype)
    return (oh - ws) + ws


def _gumbels():
    gk = jax.random.key(42)
    g = jax.random.split(gk, 5)
    lo, hi = 1e-6, 1.0 - 1e-6
    return (jax.random.uniform(g[0], (NLAYERS, 2), minval=lo, maxval=hi),
            jax.random.uniform(g[1], (NLAYERS, 3), minval=lo, maxval=hi),
            jax.random.uniform(g[2], (NLAYERS, 2), minval=lo, maxval=hi),
            jax.random.uniform(g[3], (NLAYERS, 2), minval=lo, maxval=hi),
            jax.random.uniform(g[4], (1, 4), minval=lo, maxval=hi))


def kernel(x, edge_index, pre_W1, pre_b1, pre_W2, pre_b2, comb_W, comb_b,
           jk_W, jk_b, neigh_alphas, aggr_alphas, norm_alphas, comb_alphas,
           jk_alphas):
    us = _gumbels()
    nw = _categ(neigh_alphas, us[0])
    aw = _categ(aggr_alphas, us[1])
    sw = _categ(norm_alphas, us[2])
    cw = _categ(comb_alphas, us[3])
    jw = _categ(jk_alphas, us[4])

    src = edge_index[0]
    dst = edge_index[1]
    src2d = jnp.concatenate(
        [src, jnp.zeros((EPAD - E,), jnp.int32)]).reshape(EPAD // EC, EC)
    dst2d = jnp.concatenate(
        [dst, jnp.full((EPAD - E,), N, jnp.int32)]).reshape(EPAD // EC, EC)
    srcseg = src.reshape(E // SEG, 1, SEG)
    dstseg = dst.reshape(E // SEG, 1, SEG)
    zrows = jnp.zeros((RPT, DP), jnp.float32)
    z16 = jnp.zeros((RPT, 16), jnp.float32)
    orows = jnp.ones((EC, 16), jnp.float32)

    h0 = _tc_premlp(x, pre_W1, pre_b1, pre_W2, pre_b2)
    degp = _sc_degree(dst2d, orows, z16)

    wv_prep = jnp.stack([
        (sw[0, 0] > 0.5).astype(jnp.float32),
        (aw[0, 1] > 0.5).astype(jnp.float32),
        (sw[1, 0] > 0.5).astype(jnp.float32),
        (aw[1, 1] > 0.5).astype(jnp.float32),
    ])
    scales, hs_cur = _tc_prep(degp, h0, wv_prep)

    xs = [h0]
    for l in range(NLAYERS):
        wprod = ((jnp.sum(nw[l]) * jnp.sum(aw[l])) * jnp.sum(sw[l]))
        wv_l = jnp.stack([wprod, cw[l, 0], cw[l, 1],
                          jnp.zeros((), jnp.float32)])
        emit_scaled = l + 1 < NLAYERS
        xprev = xs[-1]

        def one_hop_sum(hs=hs_cur, xp=xprev, wl=wv_l, ll=l, es=emit_scaled):
            parts = _sc_prop_sum(hs, src2d, dst2d, zrows)
            return _tc_combine(parts, xp, scales, comb_W[ll], comb_b[ll],
                               wl, ll, False, es)

        def one_hop_max(hs=hs_cur, xp=xprev, wl=wv_l, ll=l, es=emit_scaled):
            pm = _tc_segmax(hs, srcseg, dstseg)
            return _tc_combine(pm, xp, scales, comb_W[ll], comb_b[ll],
                               wl, ll, True, es)

        def two_hop_sum(hs=hs_cur, xp=xprev, wl=wv_l, ll=l, es=emit_scaled):
            p1 = _sc_prop_sum(hs, src2d, dst2d, zrows)
            hmid = _tc_mid(p1, scales, ll, False)
            p2 = _sc_prop_sum(hmid, src2d, dst2d, zrows)
            return _tc_combine(p2, xp, scales, comb_W[ll], comb_b[ll],
                               wl, ll, False, es)

        def two_hop_max(hs=hs_cur, xp=xprev, wl=wv_l, ll=l, es=emit_scaled):
            pm1 = _tc_segmax(hs, srcseg, dstseg)
            hmid = _tc_mid(pm1, scales, ll, True)
            pm2 = _tc_segmax(hmid, srcseg, dstseg)
            return _tc_combine(pm2, xp, scales, comb_W[ll], comb_b[ll],
                               wl, ll, True, es)

        bi = (2 * (nw[l, 1] > 0.5).astype(jnp.int32)
              + (aw[l, 2] > 0.5).astype(jnp.int32))
        if emit_scaled:
            xn, hs_cur = lax.switch(
                bi, [one_hop_sum, one_hop_max, two_hop_sum, two_hop_max])
        else:
            xn, _ = lax.switch(
                bi, [one_hop_sum, one_hop_max, two_hop_sum, two_hop_max])
        xs.append(xn)

    return _tc_jk(xs[0], xs[1], xs[2], jk_W, jk_b, jw[0])


# speedup vs baseline: 33.0242x; 1.1042x over previous
import functools

import jax
import jax.numpy as jnp
from jax import lax
from jax.experimental import pallas as pl
from jax.experimental.pallas import tpu as pltpu
from jax.experimental.pallas import tpu_sc as plsc

N = 10000
E = 160000
F = 128
HID = 256
C = 40
DP = 48
NLAYERS = 2
TEMP = 0.5

NC, NS = 2, 16
NW = NC * NS
EC = 128
EPAD = 163840
TPT = EPAD // NW // EC
NPAD = 10112
RPT = NPAD // NS
BR = 1000
SEG = 1600


def _sc_mesh():
    return plsc.VectorSubcoreMesh(core_axis_name="c", subcore_axis_name="s",
                                  num_cores=NC, num_subcores=NS)


def _sc_prop_sum(h_pad, src2d, dst2d, zrows):
    nbuf = 4
    nrounds = TPT // nbuf

    @functools.partial(
        pl.kernel,
        out_type=jax.ShapeDtypeStruct((NC, NPAD, DP), jnp.float32),
        mesh=_sc_mesh(),
        scratch_types=[
            pltpu.VMEM((TPT, EC), jnp.int32),
            pltpu.VMEM((TPT, EC), jnp.int32),
            [pltpu.VMEM((EC, DP), jnp.float32)] * nbuf,
            [pltpu.SemaphoreType.DMA] * nbuf,
            [pltpu.SemaphoreType.DMA] * nbuf,
            pltpu.VMEM_SHARED((NPAD, DP), jnp.float32),
        ],
        compiler_params=pltpu.CompilerParams(use_tc_tiling_on_sc=False),
    )
    def kfn(h_hbm, s_hbm, d_hbm, z_hbm, out_hbm, sidx, didx, rows, gsem,
            ssem, acc):
        c = lax.axis_index("c")
        s = lax.axis_index("s")
        pltpu.sync_copy(z_hbm, acc.at[pl.ds(s * RPT, RPT)])
        tb = (c * NS + s) * TPT
        pltpu.sync_copy(s_hbm.at[pl.ds(tb, TPT)], sidx)
        pltpu.sync_copy(d_hbm.at[pl.ds(tb, TPT)], didx)
        plsc.subcore_barrier()

        def gstart(j, b):
            pltpu.async_copy(h_hbm.at[sidx.at[j]], rows[b], gsem[b])

        def gwait(j, b):
            pltpu.make_async_copy(h_hbm.at[sidx.at[j]], rows[b],
                                  gsem[b]).wait()

        def sstart(j, b):
            pltpu.async_copy(rows[b], acc.at[didx.at[j]], ssem[b], add=True)

        def swait(j, b):
            pltpu.make_async_copy(rows[b], acc.at[didx.at[j]],
                                  ssem[b]).wait()

        for b in range(nbuf):
            gstart(b, b)

        def body(i, carry):
            j0 = i * nbuf
            for b in range(nbuf):
                gwait(j0 + b, b)
                sstart(j0 + b, b)
            for b in range(nbuf):
                swait(j0 + b, b)

                @pl.when(i < nrounds - 1)
                def _(jb=j0 + b + nbuf, bb=b):
                    gstart(jb, bb)
            return carry

        lax.fori_loop(0, nrounds, body, 0)
        plsc.subcore_barrier()
        pltpu.sync_copy(acc.at[pl.ds(s * RPT, RPT)],
                        out_hbm.at[c, pl.ds(s * RPT, RPT)])

    return kfn(h_pad, src2d, dst2d, zrows)


def _sc_degree(dst2d, orows, z16):
    @functools.partial(
        pl.kernel,
        out_type=jax.ShapeDtypeStruct((NC, NPAD, 16), jnp.float32),
        mesh=_sc_mesh(),
        scratch_types=[
            pltpu.VMEM((TPT, EC), jnp.int32),
            pltpu.VMEM((EC, 16), jnp.float32),
            pltpu.VMEM_SHARED((NPAD, 16), jnp.float32),
        ],
        compiler_params=pltpu.CompilerParams(use_tc_tiling_on_sc=False),
    )
    def kfn(d_hbm, o_hbm, z_hbm, out_hbm, didx, ones, acc):
        c = lax.axis_index("c")
        s = lax.axis_index("s")
        pltpu.sync_copy(z_hbm, acc.at[pl.ds(s * RPT, RPT)])
        pltpu.sync_copy(o_hbm, ones)
        tb = (c * NS + s) * TPT
        pltpu.sync_copy(d_hbm.at[pl.ds(tb, TPT)], didx)
        plsc.subcore_barrier()

        def body(j, carry):
            pltpu.sync_copy(ones, acc.at[didx.at[j]], add=True)
            return carry

        lax.fori_loop(0, TPT, body, 0)
        plsc.subcore_barrier()
        pltpu.sync_copy(acc.at[pl.ds(s * RPT, RPT)],
                        out_hbm.at[c, pl.ds(s * RPT, RPT)])

    return kfn(dst2d, orows, z16)


def _tc_premlp(x, W1, b1, W2, b2):
    def body(x_ref, w1_ref, b1_ref, w2_ref, b2_ref, o_ref):
        a = jnp.maximum(
            jnp.dot(x_ref[...], w1_ref[...],
                    preferred_element_type=jnp.float32) + b1_ref[...], 0.0)
        o_ref[...] = jnp.dot(a, w2_ref[...],
                             preferred_element_type=jnp.float32) + b2_ref[...]

    return pl.pallas_call(
        body,
        grid=(N // BR,),
        in_specs=[
            pl.BlockSpec((BR, F), lambda i: (i, 0)),
            pl.BlockSpec((F, HID), lambda i: (0, 0)),
            pl.BlockSpec((1, HID), lambda i: (0, 0)),
            pl.BlockSpec((HID, C), lambda i: (0, 0)),
            pl.BlockSpec((1, C), lambda i: (0, 0)),
        ],
        out_specs=pl.BlockSpec((BR, C), lambda i: (i, 0)),
        out_shape=jax.ShapeDtypeStruct((N, C), jnp.float32),
    )(x, W1, b1.reshape(1, HID), W2, b2.reshape(1, C))


def _tc_prep(degp, h, wv):
    def body(wv_ref, dp_ref, h_ref, sc_ref, hs_ref):
        deg = dp_ref[0, :, 0:1] + dp_ref[1, :, 0:1]
        pos = deg > 0.0
        dmax = jnp.maximum(deg, 1e-12)
        dis = jnp.where(pos, lax.rsqrt(dmax), 0.0)
        dinv = jnp.where(pos, 1.0 / dmax, 0.0)
        inv1 = 1.0 / jnp.maximum(deg, 1.0)
        one = jnp.ones_like(deg)
        cols = []
        for l in range(NLAYERS):
            sym = wv_ref[2 * l] > 0.5
            mean = wv_ref[2 * l + 1] > 0.5
            pre = jnp.where(sym, dis, one)
            post = jnp.where(sym, dis, dinv) * jnp.where(mean, inv1, one)
            cols += [pre, post, post * pre]
        z = jnp.zeros_like(deg)
        sc_ref[...] = jnp.concatenate(cols + [z, z], axis=1)
        hs_ref[...] = jnp.concatenate(
            [h_ref[...] * cols[0], jnp.zeros((BR, DP - C), jnp.float32)],
            axis=1)

    return pl.pallas_call(
        body,
        grid=(N // BR,),
        in_specs=[
            pl.BlockSpec(memory_space=pltpu.SMEM),
            pl.BlockSpec((NC, BR, 16), lambda i: (0, i, 0)),
            pl.BlockSpec((BR, C), lambda i: (i, 0)),
        ],
        out_specs=[
            pl.BlockSpec((BR, 8), lambda i: (i, 0)),
            pl.BlockSpec((BR, DP), lambda i: (i, 0)),
        ],
        out_shape=[
            jax.ShapeDtypeStruct((N, 8), jnp.float32),
            jax.ShapeDtypeStruct((N, DP), jnp.float32),
        ],
    )(wv, degp, h)


def _tc_segmax(hs, src2, dst2):
    def body(src_ref, dst_ref, hs_ref, o_ref):
        @pl.when(pl.program_id(0) == 0)
        def _():
            o_ref[...] = jnp.full((N, DP), -jnp.inf, jnp.float32)

        def step(e, carry):
            sv = src_ref[0, 0, e]
            dv = dst_ref[0, 0, e]
            row = hs_ref[pl.ds(sv, 1), :]
            o_ref[pl.ds(dv, 1), :] = jnp.maximum(o_ref[pl.ds(dv, 1), :], row)
            return carry

        lax.fori_loop(0, SEG, step, 0)

    return pl.pallas_call(
        body,
        grid=(E // SEG,),
        in_specs=[
            pl.BlockSpec((1, 1, SEG), lambda i: (i, 0, 0),
                         memory_space=pltpu.SMEM),
            pl.BlockSpec((1, 1, SEG), lambda i: (i, 0, 0),
                         memory_space=pltpu.SMEM),
            pl.BlockSpec((N, DP), lambda i: (0, 0)),
        ],
        out_specs=pl.BlockSpec((N, DP), lambda i: (0, 0)),
        out_shape=jax.ShapeDtypeStruct((N, DP), jnp.float32),
    )(src2, dst2, hs)


def _tc_mid(p_in, scales, l, is_max):
    mid_col = 3 * l + 2

    def body(p_ref, sc_ref, o_ref):
        if is_max:
            p48 = p_ref[...]
            p48 = jnp.where(jnp.isfinite(p48), p48, 0.0)
        else:
            p48 = p_ref[0] + p_ref[1]
        o_ref[...] = p48 * sc_ref[:, mid_col:mid_col + 1]

    p_spec = (pl.BlockSpec((BR, DP), lambda i: (i, 0)) if is_max
              else pl.BlockSpec((NC, BR, DP), lambda i: (0, i, 0)))
    return pl.pallas_call(
        body,
        grid=(N // BR,),
        in_specs=[p_spec, pl.BlockSpec((BR, 8), lambda i: (i, 0))],
        out_specs=pl.BlockSpec((BR, DP), lambda i: (i, 0)),
        out_shape=jax.ShapeDtypeStruct((N, DP), jnp.float32),
    )(p_in, scales)


def _tc_combine(p_in, xprev, scales, Wl, bl, wv, l, is_max, emit_scaled):
    post_col = 3 * l + 1

    def body(wv_ref, p_ref, xp_ref, sc_ref, w_ref, b_ref, *outs):
        if is_max:
            p48 = p_ref[...]
            p48 = jnp.where(jnp.isfinite(p48), p48, 0.0)
        else:
            p48 = p_ref[0] + p_ref[1]
        p = p48[:, :C] * sc_ref[:, post_col:post_col + 1]
        m = jnp.maximum(wv_ref[0] * p, 0.0)
        xp = xp_ref[...]
        cadd = m + xp
        ccat = (jnp.dot(m, w_ref[0:C, :], preferred_element_type=jnp.float32)
                + jnp.dot(xp, w_ref[C:, :], preferred_element_type=jnp.float32)
                + b_ref[...])
        xn = wv_ref[1] * cadd + wv_ref[2] * ccat
        outs[0][...] = xn
        if emit_scaled:
            outs[1][...] = jnp.concatenate(
                [xn * sc_ref[:, 3:4], jnp.zeros((BR, DP - C), jnp.float32)],
                axis=1)

    p_spec = (pl.BlockSpec((BR, DP), lambda i: (i, 0)) if is_max
              else pl.BlockSpec((NC, BR, DP), lambda i: (0, i, 0)))
    out_specs = [pl.BlockSpec((BR, C), lambda i: (i, 0))]
    out_shape = [jax.ShapeDtypeStruct((N, C), jnp.float32)]
    if emit_scaled:
        out_specs.append(pl.BlockSpec((BR, DP), lambda i: (i, 0)))
        out_shape.append(jax.ShapeDtypeStruct((N, DP), jnp.float32))
    res = pl.pallas_call(
        body,
        grid=(N // BR,),
        in_specs=[
            pl.BlockSpec(memory_space=pltpu.SMEM),
            p_spec,
            pl.BlockSpec((BR, C), lambda i: (i, 0)),
            pl.BlockSpec((BR, 8), lambda i: (i, 0)),
            pl.BlockSpec((2 * C, C), lambda i: (0, 0)),
            pl.BlockSpec((1, C), lambda i: (0, 0)),
        ],
        out_specs=out_specs,
        out_shape=out_shape,
    )(wv, p_in, xprev, scales, Wl, bl.reshape(1, C))
    return res if emit_scaled else (res[0], None)


def _tc_jk(h0, x1, x2, jk_W, jk_b, jw):
    def body(jw_ref, h_ref, x1_ref, x2_ref, w_ref, b_ref, o_ref):
        h = h_ref[...]
        x1 = x1_ref[...]
        x2 = x2_ref[...]
        cat = (jnp.dot(h, w_ref[0:C, :], preferred_element_type=jnp.float32)
               + jnp.dot(x1, w_ref[C:2 * C, :],
                         preferred_element_type=jnp.float32)
               + jnp.dot(x2, w_ref[2 * C:, :],
                         preferred_element_type=jnp.float32)
               + b_ref[...])
        mx = jnp.maximum(jnp.maximum(h, x1), x2)
        mn = (h + x1 + x2) / 3.0
        lin = (jw_ref[0] * x2 + jw_ref[1] * mx + jw_ref[2] * mn
               + jw_ref[3] * cat)
        rmax = jnp.max(lin, axis=1, keepdims=True)
        sh = lin - rmax
        o_ref[...] = sh - jnp.log(jnp.sum(jnp.exp(sh), axis=1, keepdims=True))

    return pl.pallas_call(
        body,
        grid=(N // BR,),
        in_specs=[
            pl.BlockSpec(memory_space=pltpu.SMEM),
            pl.BlockSpec((BR, C), lambda i: (i, 0)),
            pl.BlockSpec((BR, C), lambda i: (i, 0)),
            pl.BlockSpec((BR, C), lambda i: (i, 0)),
            pl.BlockSpec((3 * C, C), lambda i: (0, 0)),
            pl.BlockSpec((1, C), lambda i: (0, 0)),
        ],
        out_specs=pl.BlockSpec((BR, C), lambda i: (i, 0)),
        out_shape=jax.ShapeDtypeStruct((N, C), jnp.float32),
    )(jw, h0, x1, x2, jk_W, jk_b.reshape(1, C))


def _categ(alphas, u):
    ws = jax.nn.softmax((alphas - jnp.log(-jnp.log(u))) / TEMP, axis=-1)
    oh = jax.nn.one_hot(jnp.argmax(ws, axis=-1), ws.shape[-1], dtype=ws.dtype)
    return (oh - ws) + ws


def _gumbels():
    gk = jax.random.key(42)
    g = jax.random.split(gk, 5)
    lo, hi = 1e-6, 1.0 - 1e-6
    return (jax.random.uniform(g[0], (NLAYERS, 2), minval=lo, maxval=hi),
            jax.random.uniform(g[1], (NLAYERS, 3), minval=lo, maxval=hi),
            jax.random.uniform(g[2], (NLAYERS, 2), minval=lo, maxval=hi),
            jax.random.uniform(g[3], (NLAYERS, 2), minval=lo, maxval=hi),
            jax.random.uniform(g[4], (1, 4), minval=lo, maxval=hi))


def kernel(x, edge_index, pre_W1, pre_b1, pre_W2, pre_b2, comb_W, comb_b,
           jk_W, jk_b, neigh_alphas, aggr_alphas, norm_alphas, comb_alphas,
           jk_alphas):
    us = _gumbels()
    nw = _categ(neigh_alphas, us[0])
    aw = _categ(aggr_alphas, us[1])
    sw = _categ(norm_alphas, us[2])
    cw = _categ(comb_alphas, us[3])
    jw = _categ(jk_alphas, us[4])

    src = edge_index[0]
    dst = edge_index[1]
    src2d = jnp.concatenate(
        [src, jnp.zeros((EPAD - E,), jnp.int32)]).reshape(EPAD // EC, EC)
    dst2d = jnp.concatenate(
        [dst, jnp.full((EPAD - E,), N, jnp.int32)]).reshape(EPAD // EC, EC)
    srcseg = src.reshape(E // SEG, 1, SEG)
    dstseg = dst.reshape(E // SEG, 1, SEG)
    zrows = jnp.zeros((RPT, DP), jnp.float32)
    z16 = jnp.zeros((RPT, 16), jnp.float32)
    orows = jnp.ones((EC, 16), jnp.float32)

    h0 = _tc_premlp(x, pre_W1, pre_b1, pre_W2, pre_b2)
    degp = _sc_degree(dst2d, orows, z16)

    wv_prep = jnp.stack([
        (sw[0, 0] > 0.5).astype(jnp.float32),
        (aw[0, 1] > 0.5).astype(jnp.float32),
        (sw[1, 0] > 0.5).astype(jnp.float32),
        (aw[1, 1] > 0.5).astype(jnp.float32),
    ])
    scales, hs_cur = _tc_prep(degp, h0, wv_prep)

    xs = [h0]
    for l in range(NLAYERS):
        wprod = ((jnp.sum(nw[l]) * jnp.sum(aw[l])) * jnp.sum(sw[l]))
        wv_l = jnp.stack([wprod, cw[l, 0], cw[l, 1],
                          jnp.zeros((), jnp.float32)])
        emit_scaled = l + 1 < NLAYERS
        xprev = xs[-1]

        def one_hop_sum(hs=hs_cur, xp=xprev, wl=wv_l, ll=l, es=emit_scaled):
            parts = _sc_prop_sum(hs, src2d, dst2d, zrows)
            return _tc_combine(parts, xp, scales, comb_W[ll], comb_b[ll],
                               wl, ll, False, es)

        def one_hop_max(hs=hs_cur, xp=xprev, wl=wv_l, ll=l, es=emit_scaled):
            pm = _tc_segmax(hs, srcseg, dstseg)
            return _tc_combine(pm, xp, scales, comb_W[ll], comb_b[ll],
                               wl, ll, True, es)

        def two_hop_sum(hs=hs_cur, xp=xprev, wl=wv_l, ll=l, es=emit_scaled):
            p1 = _sc_prop_sum(hs, src2d, dst2d, zrows)
            hmid = _tc_mid(p1, scales, ll, False)
            p2 = _sc_prop_sum(hmid, src2d, dst2d, zrows)
            return _tc_combine(p2, xp, scales, comb_W[ll], comb_b[ll],
                               wl, ll, False, es)

        def two_hop_max(hs=hs_cur, xp=xprev, wl=wv_l, ll=l, es=emit_scaled):
            pm1 = _tc_segmax(hs, srcseg, dstseg)
            hmid = _tc_mid(pm1, scales, ll, True)
            pm2 = _tc_segmax(hmid, srcseg, dstseg)
            return _tc_combine(pm2, xp, scales, comb_W[ll], comb_b[ll],
                               wl, ll, True, es)

        bi = (2 * (nw[l, 1] > 0.5).astype(jnp.int32)
              + (aw[l, 2] > 0.5).astype(jnp.int32))
        if emit_scaled:
            xn, hs_cur = lax.switch(
                bi, [one_hop_sum, one_hop_max, two_hop_sum, two_hop_max])
        else:
            xn, _ = lax.switch(
                bi, [one_hop_sum, one_hop_max, two_hop_sum, two_hop_max])
        xs.append(xn)

    return _tc_jk(xs[0], xs[1], xs[2], jk_W, jk_b, jw[0])


# speedup vs baseline: 35.9987x; 1.0901x over previous
import functools

import jax
import jax.numpy as jnp
from jax import lax
from jax.experimental import pallas as pl
from jax.experimental.pallas import tpu as pltpu
from jax.experimental.pallas import tpu_sc as plsc

N = 10000
E = 160000
F = 128
HID = 256
C = 40
DP = 48
NLAYERS = 2
TEMP = 0.5

NC, NS = 2, 16
NW = NC * NS
EC = 128
EPAD = 163840
TPT = EPAD // NW // EC
NPAD = 10112
RPT = NPAD // NS
BR = 1000
SEG = 1600


def _sc_mesh():
    return plsc.VectorSubcoreMesh(core_axis_name="c", subcore_axis_name="s",
                                  num_cores=NC, num_subcores=NS)


def _sc_prop_sum(h_pad, src2d, dst2d, zrows):
    nbuf = 8
    nrounds = TPT // nbuf

    @functools.partial(
        pl.kernel,
        out_type=jax.ShapeDtypeStruct((NC, NPAD, DP), jnp.float32),
        mesh=_sc_mesh(),
        scratch_types=[
            pltpu.VMEM((TPT, EC), jnp.int32),
            pltpu.VMEM((TPT, EC), jnp.int32),
            [pltpu.VMEM((EC, DP), jnp.float32)] * nbuf,
            [pltpu.SemaphoreType.DMA] * nbuf,
            [pltpu.SemaphoreType.DMA] * nbuf,
            pltpu.VMEM_SHARED((NPAD, DP), jnp.float32),
        ],
        compiler_params=pltpu.CompilerParams(use_tc_tiling_on_sc=False),
    )
    def kfn(h_hbm, s_hbm, d_hbm, z_hbm, out_hbm, sidx, didx, rows, gsem,
            ssem, acc):
        c = lax.axis_index("c")
        s = lax.axis_index("s")
        pltpu.sync_copy(z_hbm, acc.at[pl.ds(s * RPT, RPT)])
        tb = (c * NS + s) * TPT
        pltpu.sync_copy(s_hbm.at[pl.ds(tb, TPT)], sidx)
        pltpu.sync_copy(d_hbm.at[pl.ds(tb, TPT)], didx)
        plsc.subcore_barrier()

        def gstart(j, b):
            pltpu.async_copy(h_hbm.at[sidx.at[j]], rows[b], gsem[b])

        def gwait(j, b):
            pltpu.make_async_copy(h_hbm.at[sidx.at[j]], rows[b],
                                  gsem[b]).wait()

        def sstart(j, b):
            pltpu.async_copy(rows[b], acc.at[didx.at[j]], ssem[b], add=True)

        def swait(j, b):
            pltpu.make_async_copy(rows[b], acc.at[didx.at[j]],
                                  ssem[b]).wait()

        for b in range(nbuf):
            gstart(b, b)

        def body(i, carry):
            j0 = i * nbuf
            for b in range(nbuf):
                gwait(j0 + b, b)
                sstart(j0 + b, b)
            for b in range(nbuf):
                swait(j0 + b, b)

                @pl.when(i < nrounds - 1)
                def _(jb=j0 + b + nbuf, bb=b):
                    gstart(jb, bb)
            return carry

        lax.fori_loop(0, nrounds, body, 0)
        plsc.subcore_barrier()
        pltpu.sync_copy(acc.at[pl.ds(s * RPT, RPT)],
                        out_hbm.at[c, pl.ds(s * RPT, RPT)])

    return kfn(h_pad, src2d, dst2d, zrows)


def _sc_degree(dst2d, orows, z16):
    @functools.partial(
        pl.kernel,
        out_type=jax.ShapeDtypeStruct((NC, NPAD, 16), jnp.float32),
        mesh=_sc_mesh(),
        scratch_types=[
            pltpu.VMEM((TPT, EC), jnp.int32),
            pltpu.VMEM((EC, 16), jnp.float32),
            pltpu.VMEM_SHARED((NPAD, 16), jnp.float32),
        ],
        compiler_params=pltpu.CompilerParams(use_tc_tiling_on_sc=False),
    )
    def kfn(d_hbm, o_hbm, z_hbm, out_hbm, didx, ones, acc):
        c = lax.axis_index("c")
        s = lax.axis_index("s")
        pltpu.sync_copy(z_hbm, acc.at[pl.ds(s * RPT, RPT)])
        pltpu.sync_copy(o_hbm, ones)
        tb = (c * NS + s) * TPT
        pltpu.sync_copy(d_hbm.at[pl.ds(tb, TPT)], didx)
        plsc.subcore_barrier()

        def body(j, carry):
            pltpu.sync_copy(ones, acc.at[didx.at[j]], add=True)
            return carry

        lax.fori_loop(0, TPT, body, 0)
        plsc.subcore_barrier()
        pltpu.sync_copy(acc.at[pl.ds(s * RPT, RPT)],
                        out_hbm.at[c, pl.ds(s * RPT, RPT)])

    return kfn(dst2d, orows, z16)


def _tc_premlp(x, W1, b1, W2, b2):
    def body(x_ref, w1_ref, b1_ref, w2_ref, b2_ref, o_ref):
        a = jnp.maximum(
            jnp.dot(x_ref[...], w1_ref[...],
                    preferred_element_type=jnp.float32) + b1_ref[...], 0.0)
        o_ref[...] = jnp.dot(a, w2_ref[...],
                             preferred_element_type=jnp.float32) + b2_ref[...]

    return pl.pallas_call(
        body,
        grid=(N // BR,),
        in_specs=[
            pl.BlockSpec((BR, F), lambda i: (i, 0)),
            pl.BlockSpec((F, HID), lambda i: (0, 0)),
            pl.BlockSpec((1, HID), lambda i: (0, 0)),
            pl.BlockSpec((HID, C), lambda i: (0, 0)),
            pl.BlockSpec((1, C), lambda i: (0, 0)),
        ],
        out_specs=pl.BlockSpec((BR, C), lambda i: (i, 0)),
        out_shape=jax.ShapeDtypeStruct((N, C), jnp.float32),
    )(x, W1, b1.reshape(1, HID), W2, b2.reshape(1, C))


def _tc_prep(degp, h, wv):
    def body(wv_ref, dp_ref, h_ref, sc_ref, hs_ref):
        deg = dp_ref[0, :, 0:1] + dp_ref[1, :, 0:1]
        pos = deg > 0.0
        dmax = jnp.maximum(deg, 1e-12)
        dis = jnp.where(pos, lax.rsqrt(dmax), 0.0)
        dinv = jnp.where(pos, 1.0 / dmax, 0.0)
        inv1 = 1.0 / jnp.maximum(deg, 1.0)
        one = jnp.ones_like(deg)
        cols = []
        for l in range(NLAYERS):
            sym = wv_ref[2 * l] > 0.5
            mean = wv_ref[2 * l + 1] > 0.5
            pre = jnp.where(sym, dis, one)
            post = jnp.where(sym, dis, dinv) * jnp.where(mean, inv1, one)
            cols += [pre, post, post * pre]
        z = jnp.zeros_like(deg)
        sc_ref[...] = jnp.concatenate(cols + [z, z], axis=1)
        hs_ref[...] = jnp.concatenate(
            [h_ref[...] * cols[0], jnp.zeros((BR, DP - C), jnp.float32)],
            axis=1)

    return pl.pallas_call(
        body,
        grid=(N // BR,),
        in_specs=[
            pl.BlockSpec(memory_space=pltpu.SMEM),
            pl.BlockSpec((NC, BR, 16), lambda i: (0, i, 0)),
            pl.BlockSpec((BR, C), lambda i: (i, 0)),
        ],
        out_specs=[
            pl.BlockSpec((BR, 8), lambda i: (i, 0)),
            pl.BlockSpec((BR, DP), lambda i: (i, 0)),
        ],
        out_shape=[
            jax.ShapeDtypeStruct((N, 8), jnp.float32),
            jax.ShapeDtypeStruct((N, DP), jnp.float32),
        ],
    )(wv, degp, h)


def _tc_segmax(hs, src2, dst2):
    def body(src_ref, dst_ref, hs_ref, o_ref):
        @pl.when(pl.program_id(0) == 0)
        def _():
            o_ref[...] = jnp.full((N, DP), -jnp.inf, jnp.float32)

        def step(e, carry):
            sv = src_ref[0, 0, e]
            dv = dst_ref[0, 0, e]
            row = hs_ref[pl.ds(sv, 1), :]
            o_ref[pl.ds(dv, 1), :] = jnp.maximum(o_ref[pl.ds(dv, 1), :], row)
            return carry

        lax.fori_loop(0, SEG, step, 0)

    return pl.pallas_call(
        body,
        grid=(E // SEG,),
        in_specs=[
            pl.BlockSpec((1, 1, SEG), lambda i: (i, 0, 0),
                         memory_space=pltpu.SMEM),
            pl.BlockSpec((1, 1, SEG), lambda i: (i, 0, 0),
                         memory_space=pltpu.SMEM),
            pl.BlockSpec((N, DP), lambda i: (0, 0)),
        ],
        out_specs=pl.BlockSpec((N, DP), lambda i: (0, 0)),
        out_shape=jax.ShapeDtypeStruct((N, DP), jnp.float32),
    )(src2, dst2, hs)


def _tc_mid(p_in, scales, l, is_max):
    mid_col = 3 * l + 2

    def body(p_ref, sc_ref, o_ref):
        if is_max:
            p48 = p_ref[...]
            p48 = jnp.where(jnp.isfinite(p48), p48, 0.0)
        else:
            p48 = p_ref[0] + p_ref[1]
        o_ref[...] = p48 * sc_ref[:, mid_col:mid_col + 1]

    p_spec = (pl.BlockSpec((BR, DP), lambda i: (i, 0)) if is_max
              else pl.BlockSpec((NC, BR, DP), lambda i: (0, i, 0)))
    return pl.pallas_call(
        body,
        grid=(N // BR,),
        in_specs=[p_spec, pl.BlockSpec((BR, 8), lambda i: (i, 0))],
        out_specs=pl.BlockSpec((BR, DP), lambda i: (i, 0)),
        out_shape=jax.ShapeDtypeStruct((N, DP), jnp.float32),
    )(p_in, scales)


def _tc_combine(p_in, xprev, scales, Wl, bl, wv, l, is_max, emit_scaled):
    post_col = 3 * l + 1

    def body(wv_ref, p_ref, xp_ref, sc_ref, w_ref, b_ref, *outs):
        if is_max:
            p48 = p_ref[...]
            p48 = jnp.where(jnp.isfinite(p48), p48, 0.0)
        else:
            p48 = p_ref[0] + p_ref[1]
        p = p48[:, :C] * sc_ref[:, post_col:post_col + 1]
        m = jnp.maximum(wv_ref[0] * p, 0.0)
        xp = xp_ref[...]
        cadd = m + xp
        ccat = (jnp.dot(m, w_ref[0:C, :], preferred_element_type=jnp.float32)
                + jnp.dot(xp, w_ref[C:, :], preferred_element_type=jnp.float32)
                + b_ref[...])
        xn = wv_ref[1] * cadd + wv_ref[2] * ccat
        outs[0][...] = xn
        if emit_scaled:
            outs[1][...] = jnp.concatenate(
                [xn * sc_ref[:, 3:4], jnp.zeros((BR, DP - C), jnp.float32)],
                axis=1)

    p_spec = (pl.BlockSpec((BR, DP), lambda i: (i, 0)) if is_max
              else pl.BlockSpec((NC, BR, DP), lambda i: (0, i, 0)))
    out_specs = [pl.BlockSpec((BR, C), lambda i: (i, 0))]
    out_shape = [jax.ShapeDtypeStruct((N, C), jnp.float32)]
    if emit_scaled:
        out_specs.append(pl.BlockSpec((BR, DP), lambda i: (i, 0)))
        out_shape.append(jax.ShapeDtypeStruct((N, DP), jnp.float32))
    res = pl.pallas_call(
        body,
        grid=(N // BR,),
        in_specs=[
            pl.BlockSpec(memory_space=pltpu.SMEM),
            p_spec,
            pl.BlockSpec((BR, C), lambda i: (i, 0)),
            pl.BlockSpec((BR, 8), lambda i: (i, 0)),
            pl.BlockSpec((2 * C, C), lambda i: (0, 0)),
            pl.BlockSpec((1, C), lambda i: (0, 0)),
        ],
        out_specs=out_specs,
        out_shape=out_shape,
    )(wv, p_in, xprev, scales, Wl, bl.reshape(1, C))
    return res if emit_scaled else (res[0], None)


def _tc_jk(h0, x1, x2, jk_W, jk_b, jw):
    def body(jw_ref, h_ref, x1_ref, x2_ref, w_ref, b_ref, o_ref):
        h = h_ref[...]
        x1 = x1_ref[...]
        x2 = x2_ref[...]
        cat = (jnp.dot(h, w_ref[0:C, :], preferred_element_type=jnp.float32)
               + jnp.dot(x1, w_ref[C:2 * C, :],
                         preferred_element_type=jnp.float32)
               + jnp.dot(x2, w_ref[2 * C:, :],
                         preferred_element_type=jnp.float32)
               + b_ref[...])
        mx = jnp.maximum(jnp.maximum(h, x1), x2)
        mn = (h + x1 + x2) / 3.0
        lin = (jw_ref[0] * x2 + jw_ref[1] * mx + jw_ref[2] * mn
               + jw_ref[3] * cat)
        rmax = jnp.max(lin, axis=1, keepdims=True)
        sh = lin - rmax
        o_ref[...] = sh - jnp.log(jnp.sum(jnp.exp(sh), axis=1, keepdims=True))

    return pl.pallas_call(
        body,
        grid=(N // BR,),
        in_specs=[
            pl.BlockSpec(memory_space=pltpu.SMEM),
            pl.BlockSpec((BR, C), lambda i: (i, 0)),
            pl.BlockSpec((BR, C), lambda i: (i, 0)),
            pl.BlockSpec((BR, C), lambda i: (i, 0)),
            pl.BlockSpec((3 * C, C), lambda i: (0, 0)),
            pl.BlockSpec((1, C), lambda i: (0, 0)),
        ],
        out_specs=pl.BlockSpec((BR, C), lambda i: (i, 0)),
        out_shape=jax.ShapeDtypeStruct((N, C), jnp.float32),
    )(jw, h0, x1, x2, jk_W, jk_b.reshape(1, C))


def _categ(alphas, u):
    ws = jax.nn.softmax((alphas - jnp.log(-jnp.log(u))) / TEMP, axis=-1)
    oh = jax.nn.one_hot(jnp.argmax(ws, axis=-1), ws.shape[-1], dtype=ws.dtype)
    return (oh - ws) + ws


def _gumbels():
    gk = jax.random.key(42)
    g = jax.random.split(gk, 5)
    lo, hi = 1e-6, 1.0 - 1e-6
    return (jax.random.uniform(g[0], (NLAYERS, 2), minval=lo, maxval=hi),
            jax.random.uniform(g[1], (NLAYERS, 3), minval=lo, maxval=hi),
            jax.random.uniform(g[2], (NLAYERS, 2), minval=lo, maxval=hi),
            jax.random.uniform(g[3], (NLAYERS, 2), minval=lo, maxval=hi),
            jax.random.uniform(g[4], (1, 4), minval=lo, maxval=hi))


def kernel(x, edge_index, pre_W1, pre_b1, pre_W2, pre_b2, comb_W, comb_b,
           jk_W, jk_b, neigh_alphas, aggr_alphas, norm_alphas, comb_alphas,
           jk_alphas):
    us = _gumbels()
    nw = _categ(neigh_alphas, us[0])
    aw = _categ(aggr_alphas, us[1])
    sw = _categ(norm_alphas, us[2])
    cw = _categ(comb_alphas, us[3])
    jw = _categ(jk_alphas, us[4])

    src = edge_index[0]
    dst = edge_index[1]
    padn = EPAD // NW - E // NW
    pad_src = jnp.zeros((NW, padn), jnp.int32)
    pad_dst = jnp.broadcast_to(
        N + (jnp.arange(padn, dtype=jnp.int32) % (NPAD - N)), (NW, padn))
    src2d = jnp.concatenate(
        [src.reshape(NW, E // NW), pad_src], axis=1).reshape(EPAD // EC, EC)
    dst2d = jnp.concatenate(
        [dst.reshape(NW, E // NW), pad_dst], axis=1).reshape(EPAD // EC, EC)
    srcseg = src.reshape(E // SEG, 1, SEG)
    dstseg = dst.reshape(E // SEG, 1, SEG)
    zrows = jnp.zeros((RPT, DP), jnp.float32)
    z16 = jnp.zeros((RPT, 16), jnp.float32)
    orows = jnp.ones((EC, 16), jnp.float32)

    h0 = _tc_premlp(x, pre_W1, pre_b1, pre_W2, pre_b2)
    degp = _sc_degree(dst2d, orows, z16)

    wv_prep = jnp.stack([
        (sw[0, 0] > 0.5).astype(jnp.float32),
        (aw[0, 1] > 0.5).astype(jnp.float32),
        (sw[1, 0] > 0.5).astype(jnp.float32),
        (aw[1, 1] > 0.5).astype(jnp.float32),
    ])
    scales, hs_cur = _tc_prep(degp, h0, wv_prep)

    xs = [h0]
    for l in range(NLAYERS):
        wprod = ((jnp.sum(nw[l]) * jnp.sum(aw[l])) * jnp.sum(sw[l]))
        wv_l = jnp.stack([wprod, cw[l, 0], cw[l, 1],
                          jnp.zeros((), jnp.float32)])
        emit_scaled = l + 1 < NLAYERS
        xprev = xs[-1]

        def one_hop_sum(hs=hs_cur, xp=xprev, wl=wv_l, ll=l, es=emit_scaled):
            parts = _sc_prop_sum(hs, src2d, dst2d, zrows)
            return _tc_combine(parts, xp, scales, comb_W[ll], comb_b[ll],
                               wl, ll, False, es)

        def one_hop_max(hs=hs_cur, xp=xprev, wl=wv_l, ll=l, es=emit_scaled):
            pm = _tc_segmax(hs, srcseg, dstseg)
            return _tc_combine(pm, xp, scales, comb_W[ll], comb_b[ll],
                               wl, ll, True, es)

        def two_hop_sum(hs=hs_cur, xp=xprev, wl=wv_l, ll=l, es=emit_scaled):
            p1 = _sc_prop_sum(hs, src2d, dst2d, zrows)
            hmid = _tc_mid(p1, scales, ll, False)
            p2 = _sc_prop_sum(hmid, src2d, dst2d, zrows)
            return _tc_combine(p2, xp, scales, comb_W[ll], comb_b[ll],
                               wl, ll, False, es)

        def two_hop_max(hs=hs_cur, xp=xprev, wl=wv_l, ll=l, es=emit_scaled):
            pm1 = _tc_segmax(hs, srcseg, dstseg)
            hmid = _tc_mid(pm1, scales, ll, True)
            pm2 = _tc_segmax(hmid, srcseg, dstseg)
            return _tc_combine(pm2, xp, scales, comb_W[ll], comb_b[ll],
                               wl, ll, True, es)

        bi = (2 * (nw[l, 1] > 0.5).astype(jnp.int32)
              + (aw[l, 2] > 0.5).astype(jnp.int32))
        if emit_scaled:
            xn, hs_cur = lax.switch(
                bi, [one_hop_sum, one_hop_max, two_hop_sum, two_hop_max])
        else:
            xn, _ = lax.switch(
                bi, [one_hop_sum, one_hop_max, two_hop_sum, two_hop_max])
        xs.append(xn)

    return _tc_jk(xs[0], xs[1], xs[2], jk_W, jk_b, jw[0])


# speedup vs baseline: 48.7054x; 1.3530x over previous
import functools

import jax
import jax.numpy as jnp
from jax import lax
from jax.experimental import pallas as pl
from jax.experimental.pallas import tpu as pltpu
from jax.experimental.pallas import tpu_sc as plsc

N = 10000
E = 160000
F = 128
HID = 256
C = 40
DP = 48
NLAYERS = 2
TEMP = 0.5

NC, NS = 2, 16
NW = NC * NS
EC = 128
EPAD = 163840
TPT = EPAD // NW // EC
NPAD = 10112
RPT = NPAD // NS
BR = 1000
SEG = 1600


def _sc_mesh():
    return plsc.VectorSubcoreMesh(core_axis_name="c", subcore_axis_name="s",
                                  num_cores=NC, num_subcores=NS)


def _sc_prop_sum(h_pad, src2d, dst2d, zrows):
    nbuf = 8
    nrounds = TPT // nbuf

    @functools.partial(
        pl.kernel,
        out_type=jax.ShapeDtypeStruct((NC, NPAD, DP), jnp.float32),
        mesh=_sc_mesh(),
        scratch_types=[
            pltpu.VMEM((TPT, EC), jnp.int32),
            pltpu.VMEM((TPT, EC), jnp.int32),
            [pltpu.VMEM((EC, DP), jnp.float32)] * nbuf,
            [pltpu.SemaphoreType.DMA] * nbuf,
            [pltpu.SemaphoreType.DMA] * nbuf,
            pltpu.VMEM_SHARED((NPAD, DP), jnp.float32),
            pltpu.VMEM_SHARED((N, DP), jnp.float32),
        ],
        compiler_params=pltpu.CompilerParams(use_tc_tiling_on_sc=False),
    )
    def kfn(h_hbm, s_hbm, d_hbm, z_hbm, out_hbm, sidx, didx, rows, gsem,
            ssem, acc, hsp):
        c = lax.axis_index("c")
        s = lax.axis_index("s")
        pltpu.sync_copy(z_hbm, acc.at[pl.ds(s * RPT, RPT)])

        @pl.when(s == 0)
        def _():
            pltpu.sync_copy(h_hbm, hsp)

        tb = (c * NS + s) * TPT
        pltpu.sync_copy(s_hbm.at[pl.ds(tb, TPT)], sidx)
        pltpu.sync_copy(d_hbm.at[pl.ds(tb, TPT)], didx)
        plsc.subcore_barrier()

        def gstart(j, b):
            pltpu.async_copy(hsp.at[sidx.at[j]], rows[b], gsem[b])

        def gwait(j, b):
            pltpu.make_async_copy(hsp.at[sidx.at[j]], rows[b],
                                  gsem[b]).wait()

        def sstart(j, b):
            pltpu.async_copy(rows[b], acc.at[didx.at[j]], ssem[b], add=True)

        def swait(j, b):
            pltpu.make_async_copy(rows[b], acc.at[didx.at[j]],
                                  ssem[b]).wait()

        for b in range(nbuf):
            gstart(b, b)

        def body(i, carry):
            j0 = i * nbuf
            for b in range(nbuf):
                gwait(j0 + b, b)
                sstart(j0 + b, b)
            for b in range(nbuf):
                swait(j0 + b, b)

                @pl.when(i < nrounds - 1)
                def _(jb=j0 + b + nbuf, bb=b):
                    gstart(jb, bb)
            return carry

        lax.fori_loop(0, nrounds, body, 0)
        plsc.subcore_barrier()
        pltpu.sync_copy(acc.at[pl.ds(s * RPT, RPT)],
                        out_hbm.at[c, pl.ds(s * RPT, RPT)])

    return kfn(h_pad, src2d, dst2d, zrows)


def _sc_degree(dst2d, orows, z16):
    @functools.partial(
        pl.kernel,
        out_type=jax.ShapeDtypeStruct((NC, NPAD, 16), jnp.float32),
        mesh=_sc_mesh(),
        scratch_types=[
            pltpu.VMEM((TPT, EC), jnp.int32),
            pltpu.VMEM((EC, 16), jnp.float32),
            pltpu.VMEM_SHARED((NPAD, 16), jnp.float32),
        ],
        compiler_params=pltpu.CompilerParams(use_tc_tiling_on_sc=False),
    )
    def kfn(d_hbm, o_hbm, z_hbm, out_hbm, didx, ones, acc):
        c = lax.axis_index("c")
        s = lax.axis_index("s")
        pltpu.sync_copy(z_hbm, acc.at[pl.ds(s * RPT, RPT)])
        pltpu.sync_copy(o_hbm, ones)
        tb = (c * NS + s) * TPT
        pltpu.sync_copy(d_hbm.at[pl.ds(tb, TPT)], didx)
        plsc.subcore_barrier()

        def body(j, carry):
            pltpu.sync_copy(ones, acc.at[didx.at[j]], add=True)
            return carry

        lax.fori_loop(0, TPT, body, 0)
        plsc.subcore_barrier()
        pltpu.sync_copy(acc.at[pl.ds(s * RPT, RPT)],
                        out_hbm.at[c, pl.ds(s * RPT, RPT)])

    return kfn(dst2d, orows, z16)


def _tc_premlp(x, W1, b1, W2, b2):
    def body(x_ref, w1_ref, b1_ref, w2_ref, b2_ref, o_ref):
        a = jnp.maximum(
            jnp.dot(x_ref[...], w1_ref[...],
                    preferred_element_type=jnp.float32) + b1_ref[...], 0.0)
        o_ref[...] = jnp.dot(a, w2_ref[...],
                             preferred_element_type=jnp.float32) + b2_ref[...]

    return pl.pallas_call(
        body,
        grid=(N // BR,),
        in_specs=[
            pl.BlockSpec((BR, F), lambda i: (i, 0)),
            pl.BlockSpec((F, HID), lambda i: (0, 0)),
            pl.BlockSpec((1, HID), lambda i: (0, 0)),
            pl.BlockSpec((HID, C), lambda i: (0, 0)),
            pl.BlockSpec((1, C), lambda i: (0, 0)),
        ],
        out_specs=pl.BlockSpec((BR, C), lambda i: (i, 0)),
        out_shape=jax.ShapeDtypeStruct((N, C), jnp.float32),
    )(x, W1, b1.reshape(1, HID), W2, b2.reshape(1, C))


def _tc_prep(degp, h, wv):
    def body(wv_ref, dp_ref, h_ref, sc_ref, hs_ref):
        deg = dp_ref[0, :, 0:1] + dp_ref[1, :, 0:1]
        pos = deg > 0.0
        dmax = jnp.maximum(deg, 1e-12)
        dis = jnp.where(pos, lax.rsqrt(dmax), 0.0)
        dinv = jnp.where(pos, 1.0 / dmax, 0.0)
        inv1 = 1.0 / jnp.maximum(deg, 1.0)
        one = jnp.ones_like(deg)
        cols = []
        for l in range(NLAYERS):
            sym = wv_ref[2 * l] > 0.5
            mean = wv_ref[2 * l + 1] > 0.5
            pre = jnp.where(sym, dis, one)
            post = jnp.where(sym, dis, dinv) * jnp.where(mean, inv1, one)
            cols += [pre, post, post * pre]
        z = jnp.zeros_like(deg)
        sc_ref[...] = jnp.concatenate(cols + [z, z], axis=1)
        hs_ref[...] = jnp.concatenate(
            [h_ref[...] * cols[0], jnp.zeros((BR, DP - C), jnp.float32)],
            axis=1)

    return pl.pallas_call(
        body,
        grid=(N // BR,),
        in_specs=[
            pl.BlockSpec(memory_space=pltpu.SMEM),
            pl.BlockSpec((NC, BR, 16), lambda i: (0, i, 0)),
            pl.BlockSpec((BR, C), lambda i: (i, 0)),
        ],
        out_specs=[
            pl.BlockSpec((BR, 8), lambda i: (i, 0)),
            pl.BlockSpec((BR, DP), lambda i: (i, 0)),
        ],
        out_shape=[
            jax.ShapeDtypeStruct((N, 8), jnp.float32),
            jax.ShapeDtypeStruct((N, DP), jnp.float32),
        ],
    )(wv, degp, h)


def _tc_segmax(hs, src2, dst2):
    def body(src_ref, dst_ref, hs_ref, o_ref):
        @pl.when(pl.program_id(0) == 0)
        def _():
            o_ref[...] = jnp.full((N, DP), -jnp.inf, jnp.float32)

        def step(e, carry):
            sv = src_ref[0, 0, e]
            dv = dst_ref[0, 0, e]
            row = hs_ref[pl.ds(sv, 1), :]
            o_ref[pl.ds(dv, 1), :] = jnp.maximum(o_ref[pl.ds(dv, 1), :], row)
            return carry

        lax.fori_loop(0, SEG, step, 0)

    return pl.pallas_call(
        body,
        grid=(E // SEG,),
        in_specs=[
            pl.BlockSpec((1, 1, SEG), lambda i: (i, 0, 0),
                         memory_space=pltpu.SMEM),
            pl.BlockSpec((1, 1, SEG), lambda i: (i, 0, 0),
                         memory_space=pltpu.SMEM),
            pl.BlockSpec((N, DP), lambda i: (0, 0)),
        ],
        out_specs=pl.BlockSpec((N, DP), lambda i: (0, 0)),
        out_shape=jax.ShapeDtypeStruct((N, DP), jnp.float32),
    )(src2, dst2, hs)


def _tc_mid(p_in, scales, l, is_max):
    mid_col = 3 * l + 2

    def body(p_ref, sc_ref, o_ref):
        if is_max:
            p48 = p_ref[...]
            p48 = jnp.where(jnp.isfinite(p48), p48, 0.0)
        else:
            p48 = p_ref[0] + p_ref[1]
        o_ref[...] = p48 * sc_ref[:, mid_col:mid_col + 1]

    p_spec = (pl.BlockSpec((BR, DP), lambda i: (i, 0)) if is_max
              else pl.BlockSpec((NC, BR, DP), lambda i: (0, i, 0)))
    return pl.pallas_call(
        body,
        grid=(N // BR,),
        in_specs=[p_spec, pl.BlockSpec((BR, 8), lambda i: (i, 0))],
        out_specs=pl.BlockSpec((BR, DP), lambda i: (i, 0)),
        out_shape=jax.ShapeDtypeStruct((N, DP), jnp.float32),
    )(p_in, scales)


def _tc_combine(p_in, xprev, scales, Wl, bl, wv, l, is_max, emit_scaled):
    post_col = 3 * l + 1

    def body(wv_ref, p_ref, xp_ref, sc_ref, w_ref, b_ref, *outs):
        if is_max:
            p48 = p_ref[...]
            p48 = jnp.where(jnp.isfinite(p48), p48, 0.0)
        else:
            p48 = p_ref[0] + p_ref[1]
        p = p48[:, :C] * sc_ref[:, post_col:post_col + 1]
        m = jnp.maximum(wv_ref[0] * p, 0.0)
        xp = xp_ref[...]
        cadd = m + xp
        ccat = (jnp.dot(m, w_ref[0:C, :], preferred_element_type=jnp.float32)
                + jnp.dot(xp, w_ref[C:, :], preferred_element_type=jnp.float32)
                + b_ref[...])
        xn = wv_ref[1] * cadd + wv_ref[2] * ccat
        outs[0][...] = xn
        if emit_scaled:
            outs[1][...] = jnp.concatenate(
                [xn * sc_ref[:, 3:4], jnp.zeros((BR, DP - C), jnp.float32)],
                axis=1)

    p_spec = (pl.BlockSpec((BR, DP), lambda i: (i, 0)) if is_max
              else pl.BlockSpec((NC, BR, DP), lambda i: (0, i, 0)))
    out_specs = [pl.BlockSpec((BR, C), lambda i: (i, 0))]
    out_shape = [jax.ShapeDtypeStruct((N, C), jnp.float32)]
    if emit_scaled:
        out_specs.append(pl.BlockSpec((BR, DP), lambda i: (i, 0)))
        out_shape.append(jax.ShapeDtypeStruct((N, DP), jnp.float32))
    res = pl.pallas_call(
        body,
        grid=(N // BR,),
        in_specs=[
            pl.BlockSpec(memory_space=pltpu.SMEM),
            p_spec,
            pl.BlockSpec((BR, C), lambda i: (i, 0)),
            pl.BlockSpec((BR, 8), lambda i: (i, 0)),
            pl.BlockSpec((2 * C, C), lambda i: (0, 0)),
            pl.BlockSpec((1, C), lambda i: (0, 0)),
        ],
        out_specs=out_specs,
        out_shape=out_shape,
    )(wv, p_in, xprev, scales, Wl, bl.reshape(1, C))
    return res if emit_scaled else (res[0], None)


def _tc_jk(h0, x1, x2, jk_W, jk_b, jw):
    def body(jw_ref, h_ref, x1_ref, x2_ref, w_ref, b_ref, o_ref):
        h = h_ref[...]
        x1 = x1_ref[...]
        x2 = x2_ref[...]
        cat = (jnp.dot(h, w_ref[0:C, :], preferred_element_type=jnp.float32)
               + jnp.dot(x1, w_ref[C:2 * C, :],
                         preferred_element_type=jnp.float32)
               + jnp.dot(x2, w_ref[2 * C:, :],
                         preferred_element_type=jnp.float32)
               + b_ref[...])
        mx = jnp.maximum(jnp.maximum(h, x1), x2)
        mn = (h + x1 + x2) / 3.0
        lin = (jw_ref[0] * x2 + jw_ref[1] * mx + jw_ref[2] * mn
               + jw_ref[3] * cat)
        rmax = jnp.max(lin, axis=1, keepdims=True)
        sh = lin - rmax
        o_ref[...] = sh - jnp.log(jnp.sum(jnp.exp(sh), axis=1, keepdims=True))

    return pl.pallas_call(
        body,
        grid=(N // BR,),
        in_specs=[
            pl.BlockSpec(memory_space=pltpu.SMEM),
            pl.BlockSpec((BR, C), lambda i: (i, 0)),
            pl.BlockSpec((BR, C), lambda i: (i, 0)),
            pl.BlockSpec((BR, C), lambda i: (i, 0)),
            pl.BlockSpec((3 * C, C), lambda i: (0, 0)),
            pl.BlockSpec((1, C), lambda i: (0, 0)),
        ],
        out_specs=pl.BlockSpec((BR, C), lambda i: (i, 0)),
        out_shape=jax.ShapeDtypeStruct((N, C), jnp.float32),
    )(jw, h0, x1, x2, jk_W, jk_b.reshape(1, C))


def _categ(alphas, u):
    ws = jax.nn.softmax((alphas - jnp.log(-jnp.log(u))) / TEMP, axis=-1)
    oh = jax.nn.one_hot(jnp.argmax(ws, axis=-1), ws.shape[-1], dtype=ws.dtype)
    return (oh - ws) + ws


def _gumbels():
    gk = jax.random.key(42)
    g = jax.random.split(gk, 5)
    lo, hi = 1e-6, 1.0 - 1e-6
    return (jax.random.uniform(g[0], (NLAYERS, 2), minval=lo, maxval=hi),
            jax.random.uniform(g[1], (NLAYERS, 3), minval=lo, maxval=hi),
            jax.random.uniform(g[2], (NLAYERS, 2), minval=lo, maxval=hi),
            jax.random.uniform(g[3], (NLAYERS, 2), minval=lo, maxval=hi),
            jax.random.uniform(g[4], (1, 4), minval=lo, maxval=hi))


def kernel(x, edge_index, pre_W1, pre_b1, pre_W2, pre_b2, comb_W, comb_b,
           jk_W, jk_b, neigh_alphas, aggr_alphas, norm_alphas, comb_alphas,
           jk_alphas):
    us = _gumbels()
    nw = _categ(neigh_alphas, us[0])
    aw = _categ(aggr_alphas, us[1])
    sw = _categ(norm_alphas, us[2])
    cw = _categ(comb_alphas, us[3])
    jw = _categ(jk_alphas, us[4])

    src = edge_index[0]
    dst = edge_index[1]
    padn = EPAD // NW - E // NW
    pad_src = jnp.zeros((NW, padn), jnp.int32)
    pad_dst = jnp.broadcast_to(
        N + (jnp.arange(padn, dtype=jnp.int32) % (NPAD - N)), (NW, padn))
    src2d = jnp.concatenate(
        [src.reshape(NW, E // NW), pad_src], axis=1).reshape(EPAD // EC, EC)
    dst2d = jnp.concatenate(
        [dst.reshape(NW, E // NW), pad_dst], axis=1).reshape(EPAD // EC, EC)
    srcseg = src.reshape(E // SEG, 1, SEG)
    dstseg = dst.reshape(E // SEG, 1, SEG)
    zrows = jnp.zeros((RPT, DP), jnp.float32)
    z16 = jnp.zeros((RPT, 16), jnp.float32)
    orows = jnp.ones((EC, 16), jnp.float32)

    h0 = _tc_premlp(x, pre_W1, pre_b1, pre_W2, pre_b2)
    degp = _sc_degree(dst2d, orows, z16)

    wv_prep = jnp.stack([
        (sw[0, 0] > 0.5).astype(jnp.float32),
        (aw[0, 1] > 0.5).astype(jnp.float32),
        (sw[1, 0] > 0.5).astype(jnp.float32),
        (aw[1, 1] > 0.5).astype(jnp.float32),
    ])
    scales, hs_cur = _tc_prep(degp, h0, wv_prep)

    xs = [h0]
    for l in range(NLAYERS):
        wprod = ((jnp.sum(nw[l]) * jnp.sum(aw[l])) * jnp.sum(sw[l]))
        wv_l = jnp.stack([wprod, cw[l, 0], cw[l, 1],
                          jnp.zeros((), jnp.float32)])
        emit_scaled = l + 1 < NLAYERS
        xprev = xs[-1]

        def one_hop_sum(hs=hs_cur, xp=xprev, wl=wv_l, ll=l, es=emit_scaled):
            parts = _sc_prop_sum(hs, src2d, dst2d, zrows)
            return _tc_combine(parts, xp, scales, comb_W[ll], comb_b[ll],
                               wl, ll, False, es)

        def one_hop_max(hs=hs_cur, xp=xprev, wl=wv_l, ll=l, es=emit_scaled):
            pm = _tc_segmax(hs, srcseg, dstseg)
            return _tc_combine(pm, xp, scales, comb_W[ll], comb_b[ll],
                               wl, ll, True, es)

        def two_hop_sum(hs=hs_cur, xp=xprev, wl=wv_l, ll=l, es=emit_scaled):
            p1 = _sc_prop_sum(hs, src2d, dst2d, zrows)
            hmid = _tc_mid(p1, scales, ll, False)
            p2 = _sc_prop_sum(hmid, src2d, dst2d, zrows)
            return _tc_combine(p2, xp, scales, comb_W[ll], comb_b[ll],
                               wl, ll, False, es)

        def two_hop_max(hs=hs_cur, xp=xprev, wl=wv_l, ll=l, es=emit_scaled):
            pm1 = _tc_segmax(hs, srcseg, dstseg)
            hmid = _tc_mid(pm1, scales, ll, True)
            pm2 = _tc_segmax(hmid, srcseg, dstseg)
            return _tc_combine(pm2, xp, scales, comb_W[ll], comb_b[ll],
                               wl, ll, True, es)

        bi = (2 * (nw[l, 1] > 0.5).astype(jnp.int32)
              + (aw[l, 2] > 0.5).astype(jnp.int32))
        if emit_scaled:
            xn, hs_cur = lax.switch(
                bi, [one_hop_sum, one_hop_max, two_hop_sum, two_hop_max])
        else:
            xn, _ = lax.switch(
                bi, [one_hop_sum, one_hop_max, two_hop_sum, two_hop_max])
        xs.append(xn)

    return _tc_jk(xs[0], xs[1], xs[2], jk_W, jk_b, jw[0])
